# Initial kernel scaffold; baseline (speedup 1.0000x reference)
import functools

import jax
import jax.numpy as jnp
from jax import lax
from jax.experimental import pallas as pl
from jax.experimental.pallas import tpu as pltpu
from jax.experimental.pallas import tpu_sc as plsc

B = 8
L = 2048
K = 16
D = 128
R = 256

NC = 2
NS = 16
NW = NC * NS
N_IDX = B * L * K
PER_W = N_IDX // NW
CH = 128
NCHUNK = PER_W // CH


def _topk_body(cq_ref, ct_ref, rot_ref, nbr_ref, eu_ref):
    b = pl.program_id(0)
    cq = cq_ref[0]
    ca = ct_ref[0]
    qx = cq[:, 0:1]
    qy = cq[:, 1:2]
    qz = cq[:, 2:3]
    ax = ca[0:1, :]
    ay = ca[1:2, :]
    az = ca[2:3, :]
    dx = qx - ax
    dy = qy - ay
    dz = qz - az
    d = dx * dx + dy * dy + dz * dz
    iota = lax.broadcasted_iota(jnp.int32, (1, L), 1)
    idxs = []
    ncx = []
    ncy = []
    ncz = []
    for _ in range(K):
        m = jnp.min(d, axis=1, keepdims=True)
        cand = jnp.where(d == m, iota, L)
        idx = jnp.min(cand, axis=1, keepdims=True)
        sel = iota == idx
        idxs.append(idx)
        ncx.append(jnp.sum(jnp.where(sel, ax, 0.0), axis=1, keepdims=True))
        ncy.append(jnp.sum(jnp.where(sel, ay, 0.0), axis=1, keepdims=True))
        ncz.append(jnp.sum(jnp.where(sel, az, 0.0), axis=1, keepdims=True))
        d = jnp.where(sel, jnp.inf, d)
    nbr = jnp.concatenate(idxs, axis=1)
    nbr_ref[0] = nbr + b * L
    nx = jnp.concatenate(ncx, axis=1)
    ny = jnp.concatenate(ncy, axis=1)
    nz = jnp.concatenate(ncz, axis=1)
    ddx = nx - qx
    ddy = ny - qy
    ddz = nz - qz
    rot = rot_ref[0]
    for r in range(3):
        e = (ddx * rot[:, r:r + 1]
             + ddy * rot[:, 3 + r:4 + r]
             + ddz * rot[:, 6 + r:7 + r])
        eu_ref[0, :, :, r] = e


def _topk_call(cq, ct, rot9, interpret=False):
    return pl.pallas_call(
        _topk_body,
        grid=(B, L // R),
        in_specs=[
            pl.BlockSpec((1, R, 3), lambda b, r: (b, r, 0)),
            pl.BlockSpec((1, 3, L), lambda b, r: (b, 0, 0)),
            pl.BlockSpec((1, R, 9), lambda b, r: (b, r, 0)),
        ],
        out_specs=[
            pl.BlockSpec((1, R, K), lambda b, r: (b, r, 0)),
            pl.BlockSpec((1, R, K, 3), lambda b, r: (b, r, 0, 0)),
        ],
        out_shape=[
            jax.ShapeDtypeStruct((B, L, K), jnp.int32),
            jax.ShapeDtypeStruct((B, L, K, 3), jnp.float32),
        ],
        interpret=interpret,
    )(cq, ct, rot9)


def _sc_gather_body(attr_hbm, gidx_hbm, out_hbm, idx_v, rows_v, sem):
    wid = lax.axis_index("s") * NC + lax.axis_index("c")
    base0 = wid * PER_W

    def body(i, carry):
        base = base0 + i * CH
        pltpu.sync_copy(gidx_hbm.at[pl.ds(base, CH)], idx_v)
        pltpu.async_copy(attr_hbm.at[idx_v], rows_v, sem).wait()
        pltpu.sync_copy(rows_v, out_hbm.at[pl.ds(base, CH)])
        return carry

    lax.fori_loop(0, NCHUNK, body, 0)


_sc_gather = functools.partial(
    pl.kernel,
    mesh=plsc.VectorSubcoreMesh(core_axis_name="c", subcore_axis_name="s"),
    out_type=jax.ShapeDtypeStruct((N_IDX, D), jnp.float32),
    scratch_types=[
        pltpu.VMEM((CH,), jnp.int32),
        pltpu.VMEM((CH, D), jnp.float32),
        pltpu.SemaphoreType.DMA,
    ],
)(_sc_gather_body)


@jax.jit
def kernel(frame, attr):
    c = frame[:, :, 0]
    ct = jnp.transpose(c, (0, 2, 1))
    rot9 = frame[:, :, 1:4].reshape(B, L, 9)
    nbr, euclid = _topk_call(c, ct, rot9)
    attr2d = attr.reshape(B * L, D)
    gidx = nbr.reshape(N_IDX)
    neigh_attr = _sc_gather(attr2d, gidx)
    return euclid, neigh_attr.reshape(B, L, K, D)


# speedup vs baseline: 9.6491x; 9.6491x over previous
import functools

import jax
import jax.numpy as jnp
from jax import lax
from jax.experimental import pallas as pl
from jax.experimental.pallas import tpu as pltpu
from jax.experimental.pallas import tpu_sc as plsc

B = 8
L = 2048
K = 16
D = 128
R = 256

NC = 2
NS = 16
NW = NC * NS
N_IDX = B * L * K
PER_W = N_IDX // NW
CH = 128
NCHUNK = PER_W // CH


def _topk_body(cq_ref, ct_ref, rot_ref, nbr_ref, eu_ref):
    b = pl.program_id(0)
    cq = cq_ref[0]
    ca = ct_ref[0]
    qx = cq[:, 0:1]
    qy = cq[:, 1:2]
    qz = cq[:, 2:3]
    ax = ca[0:1, :]
    ay = ca[1:2, :]
    az = ca[2:3, :]
    dx = qx - ax
    dy = qy - ay
    dz = qz - az
    d = dx * dx + dy * dy + dz * dz
    iota = lax.broadcasted_iota(jnp.int32, (1, L), 1)
    idxs = []
    ncx = []
    ncy = []
    ncz = []
    for _ in range(K):
        m = jnp.min(d, axis=1, keepdims=True)
        cand = jnp.where(d == m, iota, L)
        idx = jnp.min(cand, axis=1, keepdims=True)
        sel = iota == idx
        idxs.append(idx)
        ncx.append(jnp.sum(jnp.where(sel, ax, 0.0), axis=1, keepdims=True))
        ncy.append(jnp.sum(jnp.where(sel, ay, 0.0), axis=1, keepdims=True))
        ncz.append(jnp.sum(jnp.where(sel, az, 0.0), axis=1, keepdims=True))
        d = jnp.where(sel, jnp.inf, d)
    nbr = jnp.concatenate(idxs, axis=1)
    nbr_ref[0] = nbr + b * L
    nx = jnp.concatenate(ncx, axis=1)
    ny = jnp.concatenate(ncy, axis=1)
    nz = jnp.concatenate(ncz, axis=1)
    ddx = nx - qx
    ddy = ny - qy
    ddz = nz - qz
    rot = rot_ref[0]
    for r in range(3):
        e = (ddx * rot[:, r:r + 1]
             + ddy * rot[:, 3 + r:4 + r]
             + ddz * rot[:, 6 + r:7 + r])
        eu_ref[0, :, :, r] = e


def _topk_call(cq, ct, rot9, interpret=False):
    return pl.pallas_call(
        _topk_body,
        grid=(B, L // R),
        in_specs=[
            pl.BlockSpec((1, R, 3), lambda b, r: (b, r, 0)),
            pl.BlockSpec((1, 3, L), lambda b, r: (b, 0, 0)),
            pl.BlockSpec((1, R, 9), lambda b, r: (b, r, 0)),
        ],
        out_specs=[
            pl.BlockSpec((1, R, K), lambda b, r: (b, r, 0)),
            pl.BlockSpec((1, R, K, 3), lambda b, r: (b, r, 0, 0)),
        ],
        out_shape=[
            jax.ShapeDtypeStruct((B, L, K), jnp.int32),
            jax.ShapeDtypeStruct((B, L, K, 3), jnp.float32),
        ],
        interpret=interpret,
    )(cq, ct, rot9)


def _sc_gather_body(attr_hbm, gidx_hbm, out_hbm, idx_v, rows_v, sem):
    wid = lax.axis_index("s") * NC + lax.axis_index("c")
    base0 = wid * PER_W

    def body(i, carry):
        base = base0 + i * CH
        pltpu.sync_copy(gidx_hbm.at[pl.ds(base, CH)], idx_v)
        pltpu.async_copy(attr_hbm.at[idx_v], rows_v, sem).wait()
        pltpu.sync_copy(rows_v, out_hbm.at[pl.ds(base, CH)])
        return carry

    lax.fori_loop(0, NCHUNK, body, 0)


@functools.cache
def _sc_gather():
    return pl.kernel(
        _sc_gather_body,
        mesh=plsc.VectorSubcoreMesh(
            core_axis_name="c", subcore_axis_name="s", num_cores=NC),
        out_type=jax.ShapeDtypeStruct((N_IDX, D), jnp.float32),
        scratch_types=[
            pltpu.VMEM((CH,), jnp.int32),
            pltpu.VMEM((CH, D), jnp.float32),
            pltpu.SemaphoreType.DMA,
        ],
    )


@jax.jit
def kernel(frame, attr):
    c = frame[:, :, 0]
    ct = jnp.transpose(c, (0, 2, 1))
    rot9 = frame[:, :, 1:4].reshape(B, L, 9)
    nbr, euclid = _topk_call(c, ct, rot9)
    attr2d = attr.reshape(B * L, D)
    gidx = nbr.reshape(N_IDX)
    neigh_attr = _sc_gather()(attr2d, gidx)
    return euclid, neigh_attr.reshape(B, L, K, D)


# speedup vs baseline: 14.4632x; 1.4989x over previous
import functools

import jax
import jax.numpy as jnp
from jax import lax
from jax.experimental import pallas as pl
from jax.experimental.pallas import tpu as pltpu
from jax.experimental.pallas import tpu_sc as plsc

B = 8
L = 2048
K = 16
D = 128
R = 256

NC = 2
NS = 16
NW = NC * NS
N_IDX = B * L * K
QW = (B * L) // NW
PER_W = QW * K
CH = 128
NCH = PER_W // CH
NQC = QW // 16


def _topk_body(cq_ref, ct_ref, nbr_ref):
    b = pl.program_id(0)
    cq = cq_ref[0]
    ca = ct_ref[0]
    qx = cq[:, 0:1]
    qy = cq[:, 1:2]
    qz = cq[:, 2:3]
    ax = ca[0:1, :]
    ay = ca[1:2, :]
    az = ca[2:3, :]
    dx = qx - ax
    dy = qy - ay
    dz = qz - az
    d = dx * dx + dy * dy + dz * dz
    iota = lax.broadcasted_iota(jnp.int32, (1, L), 1)
    idxs = []
    for _ in range(K):
        m = jnp.min(d, axis=1, keepdims=True)
        cand = jnp.where(d == m, iota, L)
        idx = jnp.min(cand, axis=1, keepdims=True)
        idxs.append(idx)
        d = jnp.where(cand == idx, jnp.inf, d)
    nbr = jnp.concatenate(idxs, axis=1)
    nbr_ref[0] = nbr + b * L


def _topk_call(cq, ct, interpret=False):
    return pl.pallas_call(
        _topk_body,
        grid=(B, L // R),
        in_specs=[
            pl.BlockSpec((1, R, 3), lambda b, r: (b, r, 0)),
            pl.BlockSpec((1, 3, L), lambda b, r: (b, 0, 0)),
        ],
        out_specs=pl.BlockSpec((1, R, K), lambda b, r: (b, r, 0)),
        out_shape=jax.ShapeDtypeStruct((B, L, K), jnp.int32),
        interpret=interpret,
    )(cq, ct)


def _sc_body(attr_hbm, gidx_hbm, cx_hbm, cy_hbm, cz_hbm, rot_hbm,
             attr_out, eu_out,
             idx_v, rot_v, cx_v, cy_v, cz_v, eu_v, rows_a, rows_b,
             sem_a, sem_b):
    wid = lax.axis_index("s") * NC + lax.axis_index("c")
    q0 = wid * QW
    base = q0 * K
    b = wid // (NW // B)
    boff = b * L
    qloc0 = q0 - boff

    pltpu.sync_copy(gidx_hbm.at[pl.ds(base, PER_W)], idx_v)
    pltpu.sync_copy(rot_hbm.at[pl.ds(q0 * 9, QW * 9)], rot_v)
    pltpu.sync_copy(cx_hbm.at[pl.ds(boff, L)], cx_v)
    pltpu.sync_copy(cy_hbm.at[pl.ds(boff, L)], cy_v)
    pltpu.sync_copy(cz_hbm.at[pl.ds(boff, L)], cz_v)

    pltpu.async_copy(attr_hbm.at[idx_v.at[pl.ds(0, CH)]], rows_a, sem_a)

    lanes = lax.iota(jnp.int32, 16)

    def eu_chunk(i, carry):
        qoff = i * 16
        q16 = qoff + lanes
        cqx = cx_v[pl.ds(qloc0 + qoff, 16)]
        cqy = cy_v[pl.ds(qloc0 + qoff, 16)]
        cqz = cz_v[pl.ds(qloc0 + qoff, 16)]
        rot = [plsc.load_gather(rot_v, [q16 * 9 + c]) for c in range(9)]
        for k in range(K):
            gk = plsc.load_gather(idx_v, [q16 * K + k])
            jloc = gk - boff
            nx = plsc.load_gather(cx_v, [jloc])
            ny = plsc.load_gather(cy_v, [jloc])
            nz = plsc.load_gather(cz_v, [jloc])
            ddx = nx - cqx
            ddy = ny - cqy
            ddz = nz - cqz
            for r in range(3):
                e = ddx * rot[r] + ddy * rot[3 + r] + ddz * rot[6 + r]
                plsc.store_scatter(eu_v, [q16 * (K * 3) + (k * 3 + r)], e)
        return carry

    lax.fori_loop(0, NQC, eu_chunk, 0)
    pltpu.sync_copy(eu_v, eu_out.at[pl.ds(q0 * K * 3, QW * K * 3)])

    def ga_pair(m, carry):
        j = 2 * m
        pltpu.make_async_copy(
            attr_hbm.at[idx_v.at[pl.ds(j * CH, CH)]], rows_a, sem_a).wait()
        pltpu.async_copy(
            attr_hbm.at[idx_v.at[pl.ds((j + 1) * CH, CH)]], rows_b, sem_b)
        pltpu.sync_copy(rows_a, attr_out.at[pl.ds(base + j * CH, CH)])
        pltpu.make_async_copy(
            attr_hbm.at[idx_v.at[pl.ds((j + 1) * CH, CH)]], rows_b,
            sem_b).wait()

        @pl.when(m < NCH // 2 - 1)
        def _():
            pltpu.async_copy(
                attr_hbm.at[idx_v.at[pl.ds((j + 2) * CH, CH)]], rows_a, sem_a)

        pltpu.sync_copy(rows_b, attr_out.at[pl.ds(base + (j + 1) * CH, CH)])
        return carry

    lax.fori_loop(0, NCH // 2, ga_pair, 0)


@functools.cache
def _sc_call():
    return pl.kernel(
        _sc_body,
        mesh=plsc.VectorSubcoreMesh(
            core_axis_name="c", subcore_axis_name="s", num_cores=NC),
        compiler_params=pltpu.CompilerParams(needs_layout_passes=False),
        out_type=[
            jax.ShapeDtypeStruct((N_IDX, D), jnp.float32),
            jax.ShapeDtypeStruct((B * L * K * 3,), jnp.float32),
        ],
        scratch_types=[
            pltpu.VMEM((PER_W,), jnp.int32),
            pltpu.VMEM((QW * 9,), jnp.float32),
            pltpu.VMEM((L,), jnp.float32),
            pltpu.VMEM((L,), jnp.float32),
            pltpu.VMEM((L,), jnp.float32),
            pltpu.VMEM((QW * K * 3,), jnp.float32),
            pltpu.VMEM((CH, D), jnp.float32),
            pltpu.VMEM((CH, D), jnp.float32),
            pltpu.SemaphoreType.DMA,
            pltpu.SemaphoreType.DMA,
        ],
    )


@jax.jit
def kernel(frame, attr):
    c = frame[:, :, 0]
    ct = jnp.transpose(c, (0, 2, 1))
    nbr = _topk_call(c, ct)
    attr2d = attr.reshape(B * L, D)
    gidx = nbr.reshape(N_IDX)
    cx = c[:, :, 0].reshape(B * L)
    cy = c[:, :, 1].reshape(B * L)
    cz = c[:, :, 2].reshape(B * L)
    rot9 = frame[:, :, 1:4].reshape(B * L * 9)
    neigh_attr, euclid = _sc_call()(attr2d, gidx, cx, cy, cz, rot9)
    return euclid.reshape(B, L, K, 3), neigh_attr.reshape(B, L, K, D)


# speedup vs baseline: 16.8729x; 1.1666x over previous
import functools

import jax
import jax.numpy as jnp
from jax import lax
from jax.experimental import pallas as pl
from jax.experimental.pallas import tpu as pltpu
from jax.experimental.pallas import tpu_sc as plsc

B = 8
L = 2048
K = 16
D = 128
R = 256

NC = 2
NS = 16
NW = NC * NS
N_IDX = B * L * K
QW = (B * L) // NW
PER_W = QW * K
CH = 128
NCH = PER_W // CH
NQC = QW // 16


def _topk_body(cq_ref, ct_ref, nbr_ref):
    b = pl.program_id(0)
    cq = cq_ref[0]
    ca = ct_ref[0]
    qx = cq[:, 0:1]
    qy = cq[:, 1:2]
    qz = cq[:, 2:3]
    ax = ca[0:1, :]
    ay = ca[1:2, :]
    az = ca[2:3, :]
    dx = qx - ax
    dy = qy - ay
    dz = qz - az
    d = dx * dx + dy * dy + dz * dz
    iota = lax.broadcasted_iota(jnp.int32, (1, L), 1).astype(jnp.float32)
    fL = jnp.float32(L)
    idxs = []
    for k in range(K):
        m = jnp.min(d, axis=1, keepdims=True)
        cand = jnp.where(d == m, iota, fL)
        idx = jnp.min(cand, axis=1, keepdims=True)
        idxs.append(idx.astype(jnp.int32))
        if k < K - 1:
            d = jnp.where(cand == idx, jnp.inf, d)
    nbr = jnp.concatenate(idxs, axis=1)
    nbr_ref[0] = nbr + b * L


def _topk_call(cq, ct, interpret=False):
    return pl.pallas_call(
        _topk_body,
        grid=(B, L // R),
        in_specs=[
            pl.BlockSpec((1, R, 3), lambda b, r: (b, r, 0)),
            pl.BlockSpec((1, 3, L), lambda b, r: (b, 0, 0)),
        ],
        out_specs=pl.BlockSpec((1, R, K), lambda b, r: (b, r, 0)),
        out_shape=jax.ShapeDtypeStruct((B, L, K), jnp.int32),
        compiler_params=pltpu.CompilerParams(
            dimension_semantics=("parallel", "parallel")),
        interpret=interpret,
    )(cq, ct)


def _sc_body(attr_hbm, gidx_hbm, cx_hbm, cy_hbm, cz_hbm, rot_hbm,
             attr_out, eu_out,
             idx_v, rot_v, cx_v, cy_v, cz_v, eu_v, rows_a, rows_b,
             sem_a, sem_b):
    wid = lax.axis_index("s") * NC + lax.axis_index("c")
    q0 = wid * QW
    base = q0 * K
    b = wid // (NW // B)
    boff = b * L
    qloc0 = q0 - boff

    pltpu.sync_copy(gidx_hbm.at[pl.ds(base, PER_W)], idx_v)
    pltpu.sync_copy(rot_hbm.at[pl.ds(q0 * 9, QW * 9)], rot_v)
    pltpu.sync_copy(cx_hbm.at[pl.ds(boff, L)], cx_v)
    pltpu.sync_copy(cy_hbm.at[pl.ds(boff, L)], cy_v)
    pltpu.sync_copy(cz_hbm.at[pl.ds(boff, L)], cz_v)

    pltpu.async_copy(attr_hbm.at[idx_v.at[pl.ds(0, CH)]], rows_a, sem_a)

    lanes = lax.iota(jnp.int32, 16)

    def eu_chunk(i, carry):
        qoff = i * 16
        q16 = qoff + lanes
        cqx = cx_v[pl.ds(qloc0 + qoff, 16)]
        cqy = cy_v[pl.ds(qloc0 + qoff, 16)]
        cqz = cz_v[pl.ds(qloc0 + qoff, 16)]
        rot = [plsc.load_gather(rot_v, [q16 * 9 + c]) for c in range(9)]
        for k in range(K):
            gk = plsc.load_gather(idx_v, [q16 * K + k])
            jloc = gk - boff
            nx = plsc.load_gather(cx_v, [jloc])
            ny = plsc.load_gather(cy_v, [jloc])
            nz = plsc.load_gather(cz_v, [jloc])
            ddx = nx - cqx
            ddy = ny - cqy
            ddz = nz - cqz
            for r in range(3):
                e = ddx * rot[r] + ddy * rot[3 + r] + ddz * rot[6 + r]
                plsc.store_scatter(eu_v, [q16 * (K * 3) + (k * 3 + r)], e)
        return carry

    lax.fori_loop(0, NQC, eu_chunk, 0)
    pltpu.sync_copy(eu_v, eu_out.at[pl.ds(q0 * K * 3, QW * K * 3)])

    def ga_pair(m, carry):
        j = 2 * m
        pltpu.make_async_copy(
            attr_hbm.at[idx_v.at[pl.ds(j * CH, CH)]], rows_a, sem_a).wait()
        pltpu.async_copy(
            attr_hbm.at[idx_v.at[pl.ds((j + 1) * CH, CH)]], rows_b, sem_b)
        pltpu.sync_copy(rows_a, attr_out.at[pl.ds(base + j * CH, CH)])
        pltpu.make_async_copy(
            attr_hbm.at[idx_v.at[pl.ds((j + 1) * CH, CH)]], rows_b,
            sem_b).wait()

        @pl.when(m < NCH // 2 - 1)
        def _():
            pltpu.async_copy(
                attr_hbm.at[idx_v.at[pl.ds((j + 2) * CH, CH)]], rows_a, sem_a)

        pltpu.sync_copy(rows_b, attr_out.at[pl.ds(base + (j + 1) * CH, CH)])
        return carry

    lax.fori_loop(0, NCH // 2, ga_pair, 0)


@functools.cache
def _sc_call():
    return pl.kernel(
        _sc_body,
        mesh=plsc.VectorSubcoreMesh(
            core_axis_name="c", subcore_axis_name="s", num_cores=NC),
        compiler_params=pltpu.CompilerParams(needs_layout_passes=False),
        out_type=[
            jax.ShapeDtypeStruct((N_IDX, D), jnp.float32),
            jax.ShapeDtypeStruct((B * L * K * 3,), jnp.float32),
        ],
        scratch_types=[
            pltpu.VMEM((PER_W,), jnp.int32),
            pltpu.VMEM((QW * 9,), jnp.float32),
            pltpu.VMEM((L,), jnp.float32),
            pltpu.VMEM((L,), jnp.float32),
            pltpu.VMEM((L,), jnp.float32),
            pltpu.VMEM((QW * K * 3,), jnp.float32),
            pltpu.VMEM((CH, D), jnp.float32),
            pltpu.VMEM((CH, D), jnp.float32),
            pltpu.SemaphoreType.DMA,
            pltpu.SemaphoreType.DMA,
        ],
    )


@jax.jit
def kernel(frame, attr):
    c = frame[:, :, 0]
    ct = jnp.transpose(c, (0, 2, 1))
    nbr = _topk_call(c, ct)
    attr2d = attr.reshape(B * L, D)
    gidx = nbr.reshape(N_IDX)
    cx = c[:, :, 0].reshape(B * L)
    cy = c[:, :, 1].reshape(B * L)
    cz = c[:, :, 2].reshape(B * L)
    rot9 = frame[:, :, 1:4].reshape(B * L * 9)
    neigh_attr, euclid = _sc_call()(attr2d, gidx, cx, cy, cz, rot9)
    return euclid.reshape(B, L, K, 3), neigh_attr.reshape(B, L, K, D)


# speedup vs baseline: 16.8851x; 1.0007x over previous
import functools

import jax
import jax.numpy as jnp
from jax import lax
from jax.experimental import pallas as pl
from jax.experimental.pallas import tpu as pltpu
from jax.experimental.pallas import tpu_sc as plsc

B = 8
L = 2048
K = 16
D = 128
R = 512

NC = 2
NS = 16
NW = NC * NS
N_IDX = B * L * K
QW = (B * L) // NW
PER_W = QW * K
CH = 128
NCH = PER_W // CH
NQC = QW // 16


def _topk_body(cq_ref, ct_ref, nbr_ref):
    b = pl.program_id(0)
    cq = cq_ref[0]
    ca = ct_ref[0]
    qx = cq[:, 0:1]
    qy = cq[:, 1:2]
    qz = cq[:, 2:3]
    ax = ca[0:1, :]
    ay = ca[1:2, :]
    az = ca[2:3, :]
    dx = qx - ax
    dy = qy - ay
    dz = qz - az
    d = dx * dx + dy * dy + dz * dz
    iota = lax.broadcasted_iota(jnp.int32, (1, L), 1).astype(jnp.float32)
    fL = jnp.float32(L)
    idxs = []
    for k in range(K):
        m = jnp.min(d, axis=1, keepdims=True)
        cand = jnp.where(d == m, iota, fL)
        idx = jnp.min(cand, axis=1, keepdims=True)
        idxs.append(idx.astype(jnp.int32))
        if k < K - 1:
            d = jnp.where(cand == idx, jnp.inf, d)
    nbr = jnp.concatenate(idxs, axis=1)
    nbr_ref[0] = nbr + b * L


def _topk_call(cq, ct, interpret=False):
    return pl.pallas_call(
        _topk_body,
        grid=(B, L // R),
        in_specs=[
            pl.BlockSpec((1, R, 3), lambda b, r: (b, r, 0)),
            pl.BlockSpec((1, 3, L), lambda b, r: (b, 0, 0)),
        ],
        out_specs=pl.BlockSpec((1, R, K), lambda b, r: (b, r, 0)),
        out_shape=jax.ShapeDtypeStruct((B, L, K), jnp.int32),
        compiler_params=pltpu.CompilerParams(
            dimension_semantics=("parallel", "parallel")),
        interpret=interpret,
    )(cq, ct)


def _sc_body(attr_hbm, gidx_hbm, cx_hbm, cy_hbm, cz_hbm, rot_hbm,
             attr_out, eu_out,
             idx_v, rot_v, cx_v, cy_v, cz_v, eu_v, rows_a, rows_b,
             sem_a, sem_b):
    wid = lax.axis_index("s") * NC + lax.axis_index("c")
    q0 = wid * QW
    base = q0 * K
    b = wid // (NW // B)
    boff = b * L
    qloc0 = q0 - boff

    pltpu.sync_copy(gidx_hbm.at[pl.ds(base, PER_W)], idx_v)
    pltpu.sync_copy(rot_hbm.at[pl.ds(q0 * 9, QW * 9)], rot_v)
    pltpu.sync_copy(cx_hbm.at[pl.ds(boff, L)], cx_v)
    pltpu.sync_copy(cy_hbm.at[pl.ds(boff, L)], cy_v)
    pltpu.sync_copy(cz_hbm.at[pl.ds(boff, L)], cz_v)

    pltpu.async_copy(attr_hbm.at[idx_v.at[pl.ds(0, CH)]], rows_a, sem_a)

    lanes = lax.iota(jnp.int32, 16)

    def eu_chunk(i, carry):
        qoff = i * 16
        q16 = qoff + lanes
        cqx = cx_v[pl.ds(qloc0 + qoff, 16)]
        cqy = cy_v[pl.ds(qloc0 + qoff, 16)]
        cqz = cz_v[pl.ds(qloc0 + qoff, 16)]
        rot = [plsc.load_gather(rot_v, [q16 * 9 + c]) for c in range(9)]
        for k in range(K):
            gk = plsc.load_gather(idx_v, [q16 * K + k])
            jloc = gk - boff
            nx = plsc.load_gather(cx_v, [jloc])
            ny = plsc.load_gather(cy_v, [jloc])
            nz = plsc.load_gather(cz_v, [jloc])
            ddx = nx - cqx
            ddy = ny - cqy
            ddz = nz - cqz
            for r in range(3):
                e = ddx * rot[r] + ddy * rot[3 + r] + ddz * rot[6 + r]
                plsc.store_scatter(eu_v, [q16 * (K * 3) + (k * 3 + r)], e)
        return carry

    lax.fori_loop(0, NQC, eu_chunk, 0)
    pltpu.sync_copy(eu_v, eu_out.at[pl.ds(q0 * K * 3, QW * K * 3)])

    def ga_pair(m, carry):
        j = 2 * m
        pltpu.make_async_copy(
            attr_hbm.at[idx_v.at[pl.ds(j * CH, CH)]], rows_a, sem_a).wait()
        pltpu.async_copy(
            attr_hbm.at[idx_v.at[pl.ds((j + 1) * CH, CH)]], rows_b, sem_b)
        pltpu.sync_copy(rows_a, attr_out.at[pl.ds(base + j * CH, CH)])
        pltpu.make_async_copy(
            attr_hbm.at[idx_v.at[pl.ds((j + 1) * CH, CH)]], rows_b,
            sem_b).wait()

        @pl.when(m < NCH // 2 - 1)
        def _():
            pltpu.async_copy(
                attr_hbm.at[idx_v.at[pl.ds((j + 2) * CH, CH)]], rows_a, sem_a)

        pltpu.sync_copy(rows_b, attr_out.at[pl.ds(base + (j + 1) * CH, CH)])
        return carry

    lax.fori_loop(0, NCH // 2, ga_pair, 0)


@functools.cache
def _sc_call():
    return pl.kernel(
        _sc_body,
        mesh=plsc.VectorSubcoreMesh(
            core_axis_name="c", subcore_axis_name="s", num_cores=NC),
        compiler_params=pltpu.CompilerParams(needs_layout_passes=False),
        out_type=[
            jax.ShapeDtypeStruct((N_IDX, D), jnp.float32),
            jax.ShapeDtypeStruct((B * L * K * 3,), jnp.float32),
        ],
        scratch_types=[
            pltpu.VMEM((PER_W,), jnp.int32),
            pltpu.VMEM((QW * 9,), jnp.float32),
            pltpu.VMEM((L,), jnp.float32),
            pltpu.VMEM((L,), jnp.float32),
            pltpu.VMEM((L,), jnp.float32),
            pltpu.VMEM((QW * K * 3,), jnp.float32),
            pltpu.VMEM((CH, D), jnp.float32),
            pltpu.VMEM((CH, D), jnp.float32),
            pltpu.SemaphoreType.DMA,
            pltpu.SemaphoreType.DMA,
        ],
    )


@jax.jit
def kernel(frame, attr):
    c = frame[:, :, 0]
    ct = jnp.transpose(c, (0, 2, 1))
    nbr = _topk_call(c, ct)
    attr2d = attr.reshape(B * L, D)
    gidx = nbr.reshape(N_IDX)
    cx = c[:, :, 0].reshape(B * L)
    cy = c[:, :, 1].reshape(B * L)
    cz = c[:, :, 2].reshape(B * L)
    rot9 = frame[:, :, 1:4].reshape(B * L * 9)
    neigh_attr, euclid = _sc_call()(attr2d, gidx, cx, cy, cz, rot9)
    return euclid.reshape(B, L, K, 3), neigh_attr.reshape(B, L, K, D)


# speedup vs baseline: 17.2377x; 1.0209x over previous
import functools

import jax
import jax.numpy as jnp
from jax import lax
from jax.experimental import pallas as pl
from jax.experimental.pallas import tpu as pltpu
from jax.experimental.pallas import tpu_sc as plsc

B = 8
L = 2048
K = 16
D = 128
R = 512

NC = 2
NS = 16
NW = NC * NS
N_IDX = B * L * K
QW = (B * L) // NW
PER_W = QW * K
CH = 128
NCH = PER_W // CH
NQC = QW // 16


def _topk_body(cq_ref, ct_ref, nbr_ref):
    b = pl.program_id(0)
    cq = cq_ref[0]
    ca = ct_ref[0]
    qx = cq[:, 0:1]
    qy = cq[:, 1:2]
    qz = cq[:, 2:3]
    ax = ca[0:1, :]
    ay = ca[1:2, :]
    az = ca[2:3, :]
    dx = qx - ax
    dy = qy - ay
    dz = qz - az
    d = dx * dx + dy * dy + dz * dz
    iota = lax.broadcasted_iota(jnp.int32, (1, L), 1).astype(jnp.float32)
    fL = jnp.float32(L)
    rowloc = (lax.broadcasted_iota(jnp.int32, (R, 1), 0)
              + pl.program_id(1) * R)
    d = jnp.where(iota == rowloc.astype(jnp.float32), jnp.inf, d)
    idxs = [rowloc]
    for k in range(1, K):
        m = jnp.min(d, axis=1, keepdims=True)
        cand = jnp.where(d == m, iota, fL)
        idx = jnp.min(cand, axis=1, keepdims=True)
        idxs.append(idx.astype(jnp.int32))
        if k < K - 1:
            d = jnp.where(cand == idx, jnp.inf, d)
    nbr = jnp.concatenate(idxs, axis=1)
    nbr_ref[0] = nbr + b * L


def _topk_call(cq, ct, interpret=False):
    return pl.pallas_call(
        _topk_body,
        grid=(B, L // R),
        in_specs=[
            pl.BlockSpec((1, R, 3), lambda b, r: (b, r, 0)),
            pl.BlockSpec((1, 3, L), lambda b, r: (b, 0, 0)),
        ],
        out_specs=pl.BlockSpec((1, R, K), lambda b, r: (b, r, 0)),
        out_shape=jax.ShapeDtypeStruct((B, L, K), jnp.int32),
        compiler_params=pltpu.CompilerParams(
            dimension_semantics=("parallel", "parallel")),
        interpret=interpret,
    )(cq, ct)


def _sc_body(attr_hbm, gidx_hbm, cx_hbm, cy_hbm, cz_hbm, rot_hbm,
             attr_out, eu_out,
             idx_v, rot_v, cx_v, cy_v, cz_v, eu_v, rows_a, rows_b,
             sem_a, sem_b):
    wid = lax.axis_index("s") * NC + lax.axis_index("c")
    q0 = wid * QW
    base = q0 * K
    b = wid // (NW // B)
    boff = b * L
    qloc0 = q0 - boff

    pltpu.sync_copy(gidx_hbm.at[pl.ds(base, PER_W)], idx_v)
    pltpu.sync_copy(rot_hbm.at[pl.ds(q0 * 9, QW * 9)], rot_v)
    pltpu.sync_copy(cx_hbm.at[pl.ds(boff, L)], cx_v)
    pltpu.sync_copy(cy_hbm.at[pl.ds(boff, L)], cy_v)
    pltpu.sync_copy(cz_hbm.at[pl.ds(boff, L)], cz_v)

    pltpu.async_copy(attr_hbm.at[idx_v.at[pl.ds(0, CH)]], rows_a, sem_a)

    lanes = lax.iota(jnp.int32, 16)

    def eu_chunk(i, carry):
        qoff = i * 16
        q16 = qoff + lanes
        cqx = cx_v[pl.ds(qloc0 + qoff, 16)]
        cqy = cy_v[pl.ds(qloc0 + qoff, 16)]
        cqz = cz_v[pl.ds(qloc0 + qoff, 16)]
        rot = [plsc.load_gather(rot_v, [q16 * 9 + c]) for c in range(9)]
        for k in range(K):
            gk = plsc.load_gather(idx_v, [q16 * K + k])
            jloc = gk - boff
            nx = plsc.load_gather(cx_v, [jloc])
            ny = plsc.load_gather(cy_v, [jloc])
            nz = plsc.load_gather(cz_v, [jloc])
            ddx = nx - cqx
            ddy = ny - cqy
            ddz = nz - cqz
            for r in range(3):
                e = ddx * rot[r] + ddy * rot[3 + r] + ddz * rot[6 + r]
                plsc.store_scatter(eu_v, [q16 * (K * 3) + (k * 3 + r)], e)
        return carry

    lax.fori_loop(0, NQC, eu_chunk, 0)
    pltpu.sync_copy(eu_v, eu_out.at[pl.ds(q0 * K * 3, QW * K * 3)])

    def ga_pair(m, carry):
        j = 2 * m
        pltpu.make_async_copy(
            attr_hbm.at[idx_v.at[pl.ds(j * CH, CH)]], rows_a, sem_a).wait()
        pltpu.async_copy(
            attr_hbm.at[idx_v.at[pl.ds((j + 1) * CH, CH)]], rows_b, sem_b)
        pltpu.sync_copy(rows_a, attr_out.at[pl.ds(base + j * CH, CH)])
        pltpu.make_async_copy(
            attr_hbm.at[idx_v.at[pl.ds((j + 1) * CH, CH)]], rows_b,
            sem_b).wait()

        @pl.when(m < NCH // 2 - 1)
        def _():
            pltpu.async_copy(
                attr_hbm.at[idx_v.at[pl.ds((j + 2) * CH, CH)]], rows_a, sem_a)

        pltpu.sync_copy(rows_b, attr_out.at[pl.ds(base + (j + 1) * CH, CH)])
        return carry

    lax.fori_loop(0, NCH // 2, ga_pair, 0)


@functools.cache
def _sc_call():
    return pl.kernel(
        _sc_body,
        mesh=plsc.VectorSubcoreMesh(
            core_axis_name="c", subcore_axis_name="s", num_cores=NC),
        compiler_params=pltpu.CompilerParams(needs_layout_passes=False),
        out_type=[
            jax.ShapeDtypeStruct((N_IDX, D), jnp.float32),
            jax.ShapeDtypeStruct((B * L * K * 3,), jnp.float32),
        ],
        scratch_types=[
            pltpu.VMEM((PER_W,), jnp.int32),
            pltpu.VMEM((QW * 9,), jnp.float32),
            pltpu.VMEM((L,), jnp.float32),
            pltpu.VMEM((L,), jnp.float32),
            pltpu.VMEM((L,), jnp.float32),
            pltpu.VMEM((QW * K * 3,), jnp.float32),
            pltpu.VMEM((CH, D), jnp.float32),
            pltpu.VMEM((CH, D), jnp.float32),
            pltpu.SemaphoreType.DMA,
            pltpu.SemaphoreType.DMA,
        ],
    )


@jax.jit
def kernel(frame, attr):
    c = frame[:, :, 0]
    ct = jnp.transpose(c, (0, 2, 1))
    nbr = _topk_call(c, ct)
    attr2d = attr.reshape(B * L, D)
    gidx = nbr.reshape(N_IDX)
    cx = c[:, :, 0].reshape(B * L)
    cy = c[:, :, 1].reshape(B * L)
    cz = c[:, :, 2].reshape(B * L)
    rot9 = frame[:, :, 1:4].reshape(B * L * 9)
    neigh_attr, euclid = _sc_call()(attr2d, gidx, cx, cy, cz, rot9)
    return euclid.reshape(B, L, K, 3), neigh_attr.reshape(B, L, K, D)


# speedup vs baseline: 18.3534x; 1.0647x over previous
import functools

import jax
import jax.numpy as jnp
from jax import lax
from jax.experimental import pallas as pl
from jax.experimental.pallas import tpu as pltpu
from jax.experimental.pallas import tpu_sc as plsc

B = 8
L = 2048
K = 16
D = 128
R = 512

NC = 2
NS = 16
NW = NC * NS
N_IDX = B * L * K
CH = 128

HB = B // 2
HQ = HB * L
HN = HQ * K
QW = HQ // NW
PER_W = QW * K
NCH = PER_W // CH
NQC = QW // 16


def _topk_body(b0, cq_ref, ct_ref, nbr_ref):
    b = pl.program_id(0)
    cq = cq_ref[0]
    ca = ct_ref[0]
    qx = cq[:, 0:1]
    qy = cq[:, 1:2]
    qz = cq[:, 2:3]
    ax = ca[0:1, :]
    ay = ca[1:2, :]
    az = ca[2:3, :]
    dx = qx - ax
    dy = qy - ay
    dz = qz - az
    d = dx * dx + dy * dy + dz * dz
    iota = lax.broadcasted_iota(jnp.int32, (1, L), 1).astype(jnp.float32)
    fL = jnp.float32(L)
    rowloc = (lax.broadcasted_iota(jnp.int32, (R, 1), 0)
              + pl.program_id(1) * R)
    d = jnp.where(iota == rowloc.astype(jnp.float32), jnp.inf, d)
    idxs = [rowloc]
    for k in range(1, K):
        m = jnp.min(d, axis=1, keepdims=True)
        cand = jnp.where(d == m, iota, fL)
        idx = jnp.min(cand, axis=1, keepdims=True)
        idxs.append(idx.astype(jnp.int32))
        if k < K - 1:
            d = jnp.where(cand == idx, jnp.inf, d)
    nbr = jnp.concatenate(idxs, axis=1)
    nbr_ref[0] = nbr + (b + b0) * L


def _topk_call(cq, ct, b0):
    return pl.pallas_call(
        functools.partial(_topk_body, b0),
        grid=(HB, L // R),
        in_specs=[
            pl.BlockSpec((1, R, 3), lambda b, r: (b, r, 0)),
            pl.BlockSpec((1, 3, L), lambda b, r: (b, 0, 0)),
        ],
        out_specs=pl.BlockSpec((1, R, K), lambda b, r: (b, r, 0)),
        out_shape=jax.ShapeDtypeStruct((HB, L, K), jnp.int32),
        compiler_params=pltpu.CompilerParams(
            dimension_semantics=("parallel", "parallel")),
    )(cq, ct)


def _sc_body(h, attr_hbm, gidx_hbm, cx_hbm, cy_hbm, cz_hbm, rot_hbm,
             attr_out, eu_out,
             idx_v, rot_v, cx_v, cy_v, cz_v, eu_v, rows_a, rows_b,
             sem_a, sem_b):
    wid = lax.axis_index("s") * NC + lax.axis_index("c")
    base_in = wid * PER_W
    q0 = h * HQ + wid * QW
    base_out = q0 * K
    b = q0 // L
    boff = b * L
    qloc0 = q0 - boff

    pltpu.sync_copy(gidx_hbm.at[pl.ds(base_in, PER_W)], idx_v)
    pltpu.sync_copy(rot_hbm.at[pl.ds(q0 * 9, QW * 9)], rot_v)
    pltpu.sync_copy(cx_hbm.at[pl.ds(boff, L)], cx_v)
    pltpu.sync_copy(cy_hbm.at[pl.ds(boff, L)], cy_v)
    pltpu.sync_copy(cz_hbm.at[pl.ds(boff, L)], cz_v)

    pltpu.async_copy(attr_hbm.at[idx_v.at[pl.ds(0, CH)]], rows_a, sem_a)

    lanes = lax.iota(jnp.int32, 16)

    def eu_chunk(i, carry):
        qoff = i * 16
        q16 = qoff + lanes
        cqx = cx_v[pl.ds(qloc0 + qoff, 16)]
        cqy = cy_v[pl.ds(qloc0 + qoff, 16)]
        cqz = cz_v[pl.ds(qloc0 + qoff, 16)]
        rot = [plsc.load_gather(rot_v, [q16 * 9 + c]) for c in range(9)]
        for k in range(K):
            gk = plsc.load_gather(idx_v, [q16 * K + k])
            jloc = gk - boff
            nx = plsc.load_gather(cx_v, [jloc])
            ny = plsc.load_gather(cy_v, [jloc])
            nz = plsc.load_gather(cz_v, [jloc])
            ddx = nx - cqx
            ddy = ny - cqy
            ddz = nz - cqz
            for r in range(3):
                e = ddx * rot[r] + ddy * rot[3 + r] + ddz * rot[6 + r]
                plsc.store_scatter(eu_v, [q16 * (K * 3) + (k * 3 + r)], e)
        return carry

    lax.fori_loop(0, NQC, eu_chunk, 0)
    pltpu.sync_copy(eu_v, eu_out.at[pl.ds(q0 * K * 3, QW * K * 3)])

    def ga_pair(m, carry):
        j = 2 * m
        pltpu.make_async_copy(
            attr_hbm.at[idx_v.at[pl.ds(j * CH, CH)]], rows_a, sem_a).wait()
        pltpu.async_copy(
            attr_hbm.at[idx_v.at[pl.ds((j + 1) * CH, CH)]], rows_b, sem_b)
        pltpu.sync_copy(rows_a, attr_out.at[pl.ds(base_out + j * CH, CH)])
        pltpu.make_async_copy(
            attr_hbm.at[idx_v.at[pl.ds((j + 1) * CH, CH)]], rows_b,
            sem_b).wait()

        @pl.when(m < NCH // 2 - 1)
        def _():
            pltpu.async_copy(
                attr_hbm.at[idx_v.at[pl.ds((j + 2) * CH, CH)]], rows_a, sem_a)

        pltpu.sync_copy(
            rows_b, attr_out.at[pl.ds(base_out + (j + 1) * CH, CH)])
        return carry

    lax.fori_loop(0, NCH // 2, ga_pair, 0)


@functools.cache
def _sc_call(h):
    return pl.kernel(
        functools.partial(_sc_body, h),
        mesh=plsc.VectorSubcoreMesh(
            core_axis_name="c", subcore_axis_name="s", num_cores=NC),
        compiler_params=pltpu.CompilerParams(needs_layout_passes=False),
        out_type=(),
        scratch_types=[
            pltpu.VMEM((PER_W,), jnp.int32),
            pltpu.VMEM((QW * 9,), jnp.float32),
            pltpu.VMEM((L,), jnp.float32),
            pltpu.VMEM((L,), jnp.float32),
            pltpu.VMEM((L,), jnp.float32),
            pltpu.VMEM((QW * K * 3,), jnp.float32),
            pltpu.VMEM((CH, D), jnp.float32),
            pltpu.VMEM((CH, D), jnp.float32),
            pltpu.SemaphoreType.DMA,
            pltpu.SemaphoreType.DMA,
        ],
    )


@jax.jit
def kernel(frame, attr):
    c = frame[:, :, 0]
    ct = jnp.transpose(c, (0, 2, 1))
    attr2d = attr.reshape(B * L, D)
    cx = c[:, :, 0].reshape(B * L)
    cy = c[:, :, 1].reshape(B * L)
    cz = c[:, :, 2].reshape(B * L)
    rot9 = frame[:, :, 1:4].reshape(B * L * 9)
    attr_ref = jax.empty_ref(
        jax.ShapeDtypeStruct((N_IDX, D), jnp.float32))
    eu_ref = jax.empty_ref(
        jax.ShapeDtypeStruct((B * L * K * 3,), jnp.float32))
    for h in range(2):
        s = slice(h * HB, (h + 1) * HB)
        nbr = _topk_call(c[s], ct[s], h * HB)
        _sc_call(h)(attr2d, nbr.reshape(HN), cx, cy, cz, rot9,
                    attr_ref, eu_ref)
    euclid = jax.freeze(eu_ref)
    neigh_attr = jax.freeze(attr_ref)
    return euclid.reshape(B, L, K, 3), neigh_attr.reshape(B, L, K, D)


# speedup vs baseline: 18.9806x; 1.0342x over previous
import functools

import jax
import jax.numpy as jnp
from jax import lax
from jax.experimental import pallas as pl
from jax.experimental.pallas import tpu as pltpu
from jax.experimental.pallas import tpu_sc as plsc

B = 8
L = 2048
K = 16
D = 128
R = 512

NC = 2
NS = 16
NW = NC * NS
N_IDX = B * L * K
CH = 128

HB = B // 4
HQ = HB * L
HN = HQ * K
QW = HQ // NW
PER_W = QW * K
NCH = PER_W // CH
NQC = QW // 16


def _topk_body(b0, cq_ref, ct_ref, nbr_ref):
    b = pl.program_id(0)
    cq = cq_ref[0]
    ca = ct_ref[0]
    qx = cq[:, 0:1]
    qy = cq[:, 1:2]
    qz = cq[:, 2:3]
    ax = ca[0:1, :]
    ay = ca[1:2, :]
    az = ca[2:3, :]
    dx = qx - ax
    dy = qy - ay
    dz = qz - az
    d = dx * dx + dy * dy + dz * dz
    iota = lax.broadcasted_iota(jnp.int32, (1, L), 1).astype(jnp.float32)
    fL = jnp.float32(L)
    rowloc = (lax.broadcasted_iota(jnp.int32, (R, 1), 0)
              + pl.program_id(1) * R)
    d = jnp.where(iota == rowloc.astype(jnp.float32), jnp.inf, d)
    idxs = [rowloc]
    for k in range(1, K):
        m = jnp.min(d, axis=1, keepdims=True)
        cand = jnp.where(d == m, iota, fL)
        idx = jnp.min(cand, axis=1, keepdims=True)
        idxs.append(idx.astype(jnp.int32))
        if k < K - 1:
            d = jnp.where(cand == idx, jnp.inf, d)
    nbr = jnp.concatenate(idxs, axis=1)
    nbr_ref[0] = nbr + (b + b0) * L


def _topk_call(cq, ct, b0):
    return pl.pallas_call(
        functools.partial(_topk_body, b0),
        grid=(HB, L // R),
        in_specs=[
            pl.BlockSpec((1, R, 3), lambda b, r: (b, r, 0)),
            pl.BlockSpec((1, 3, L), lambda b, r: (b, 0, 0)),
        ],
        out_specs=pl.BlockSpec((1, R, K), lambda b, r: (b, r, 0)),
        out_shape=jax.ShapeDtypeStruct((HB, L, K), jnp.int32),
        compiler_params=pltpu.CompilerParams(
            dimension_semantics=("parallel", "parallel")),
    )(cq, ct)


def _sc_body(h, attr_hbm, gidx_hbm, cx_hbm, cy_hbm, cz_hbm, rot_hbm,
             attr_out, eu_out,
             idx_v, rot_v, cx_v, cy_v, cz_v, eu_v, rows_a, rows_b,
             sem_a, sem_b):
    wid = lax.axis_index("s") * NC + lax.axis_index("c")
    base_in = wid * PER_W
    q0 = h * HQ + wid * QW
    base_out = q0 * K
    b = q0 // L
    boff = b * L
    qloc0 = q0 - boff

    pltpu.sync_copy(gidx_hbm.at[pl.ds(base_in, PER_W)], idx_v)
    pltpu.sync_copy(rot_hbm.at[pl.ds(q0 * 9, QW * 9)], rot_v)
    pltpu.sync_copy(cx_hbm.at[pl.ds(boff, L)], cx_v)
    pltpu.sync_copy(cy_hbm.at[pl.ds(boff, L)], cy_v)
    pltpu.sync_copy(cz_hbm.at[pl.ds(boff, L)], cz_v)

    pltpu.async_copy(attr_hbm.at[idx_v.at[pl.ds(0, CH)]], rows_a, sem_a)

    lanes = lax.iota(jnp.int32, 16)

    def eu_chunk(i, carry):
        qoff = i * 16
        q16 = qoff + lanes
        cqx = cx_v[pl.ds(qloc0 + qoff, 16)]
        cqy = cy_v[pl.ds(qloc0 + qoff, 16)]
        cqz = cz_v[pl.ds(qloc0 + qoff, 16)]
        rot = [plsc.load_gather(rot_v, [q16 * 9 + c]) for c in range(9)]
        for k in range(K):
            gk = plsc.load_gather(idx_v, [q16 * K + k])
            jloc = gk - boff
            nx = plsc.load_gather(cx_v, [jloc])
            ny = plsc.load_gather(cy_v, [jloc])
            nz = plsc.load_gather(cz_v, [jloc])
            ddx = nx - cqx
            ddy = ny - cqy
            ddz = nz - cqz
            for r in range(3):
                e = ddx * rot[r] + ddy * rot[3 + r] + ddz * rot[6 + r]
                plsc.store_scatter(eu_v, [q16 * (K * 3) + (k * 3 + r)], e)
        return carry

    lax.fori_loop(0, NQC, eu_chunk, 0)
    pltpu.sync_copy(eu_v, eu_out.at[pl.ds(q0 * K * 3, QW * K * 3)])

    def ga_pair(m, carry):
        j = 2 * m
        pltpu.make_async_copy(
            attr_hbm.at[idx_v.at[pl.ds(j * CH, CH)]], rows_a, sem_a).wait()
        pltpu.async_copy(
            attr_hbm.at[idx_v.at[pl.ds((j + 1) * CH, CH)]], rows_b, sem_b)
        pltpu.sync_copy(rows_a, attr_out.at[pl.ds(base_out + j * CH, CH)])
        pltpu.make_async_copy(
            attr_hbm.at[idx_v.at[pl.ds((j + 1) * CH, CH)]], rows_b,
            sem_b).wait()

        @pl.when(m < NCH // 2 - 1)
        def _():
            pltpu.async_copy(
                attr_hbm.at[idx_v.at[pl.ds((j + 2) * CH, CH)]], rows_a, sem_a)

        pltpu.sync_copy(
            rows_b, attr_out.at[pl.ds(base_out + (j + 1) * CH, CH)])
        return carry

    lax.fori_loop(0, NCH // 2, ga_pair, 0)


@functools.cache
def _sc_call(h):
    return pl.kernel(
        functools.partial(_sc_body, h),
        mesh=plsc.VectorSubcoreMesh(
            core_axis_name="c", subcore_axis_name="s", num_cores=NC),
        compiler_params=pltpu.CompilerParams(needs_layout_passes=False),
        out_type=(),
        scratch_types=[
            pltpu.VMEM((PER_W,), jnp.int32),
            pltpu.VMEM((QW * 9,), jnp.float32),
            pltpu.VMEM((L,), jnp.float32),
            pltpu.VMEM((L,), jnp.float32),
            pltpu.VMEM((L,), jnp.float32),
            pltpu.VMEM((QW * K * 3,), jnp.float32),
            pltpu.VMEM((CH, D), jnp.float32),
            pltpu.VMEM((CH, D), jnp.float32),
            pltpu.SemaphoreType.DMA,
            pltpu.SemaphoreType.DMA,
        ],
    )


@jax.jit
def kernel(frame, attr):
    c = frame[:, :, 0]
    ct = jnp.transpose(c, (0, 2, 1))
    attr2d = attr.reshape(B * L, D)
    cx = c[:, :, 0].reshape(B * L)
    cy = c[:, :, 1].reshape(B * L)
    cz = c[:, :, 2].reshape(B * L)
    rot9 = frame[:, :, 1:4].reshape(B * L * 9)
    attr_ref = jax.empty_ref(
        jax.ShapeDtypeStruct((N_IDX, D), jnp.float32))
    eu_ref = jax.empty_ref(
        jax.ShapeDtypeStruct((B * L * K * 3,), jnp.float32))
    for h in range(4):
        s = slice(h * HB, (h + 1) * HB)
        nbr = _topk_call(c[s], ct[s], h * HB)
        _sc_call(h)(attr2d, nbr.reshape(HN), cx, cy, cz, rot9,
                    attr_ref, eu_ref)
    euclid = jax.freeze(eu_ref)
    neigh_attr = jax.freeze(attr_ref)
    return euclid.reshape(B, L, K, 3), neigh_attr.reshape(B, L, K, D)
